# baseline (speedup 1.0000x reference)
import jax
import jax.numpy as jnp
from jax import lax
from jax.experimental import pallas as pl
from jax.experimental.pallas import tpu as pltpu
from jax.experimental.pallas import tpu_sc as plsc

N_NODES = 10000
N_PAD = 10240
N_EDGES = 160000
NCORE = 2
NSUB = 16
CHUNK = 200
EDGES_PER_SUB = N_EDGES // NSUB
NCHUNK = EDGES_PER_SUB // CHUNK
NBUF = 3
PRE = min(NBUF - 1, NCHUNK)
MAIN = ((NCHUNK - PRE) // NBUF) * NBUF
ROWS_PER_SUB = N_PAD // NSUB
DIN = 256
DSLAB = 64
DY = 32

_MESH = plsc.VectorSubcoreMesh(core_axis_name="c", subcore_axis_name="s",
                               num_cores=NCORE, num_subcores=NSUB)


def _ring_loop(gather_slice, srcv, bufs, sems, scat):
    def fire(j, b):
        pltpu.async_copy(gather_slice(j), bufs[b], sems[b])

    def wait(j, b):
        pltpu.make_async_copy(gather_slice(j), bufs[b], sems[b]).wait()

    for j in range(PRE):
        fire(j, j % NBUF)

    def body(i, carry):
        for b in range(NBUF):
            j = NBUF * i + b
            wait(j, b)
            fire(j + PRE, (b + PRE) % NBUF)
            scat(j, bufs[b])
        return carry
    lax.fori_loop(0, MAIN // NBUF, body, 0)

    for j in range(MAIN, NCHUNK):
        wait(j, j % NBUF)
        if j + PRE < NCHUNK:
            fire(j + PRE, (j + PRE) % NBUF)
        scat(j, bufs[j % NBUF])


def _agg1_body(x4, src4q, dst3, zer64, zer16, ones16,
               z, recip16,
               acc, deg, srcv, dstv, buf0, buf1, buf2, onesv, degbuf,
               sem0, sem1, sem2):
    c = lax.axis_index("c")
    s = lax.axis_index("s")
    row0 = s * ROWS_PER_SUB
    stripe = pl.ds(row0, ROWS_PER_SUB)

    pltpu.sync_copy(ones16, onesv)
    pltpu.sync_copy(dst3.at[s], dstv)

    def edge_loop(with_deg):
        def gather_slice(j):
            return x4.at[srcv.at[j]]

        def scat(j, buf):
            pltpu.sync_copy(buf, acc.at[dstv.at[j]], add=True)
            if with_deg:
                pltpu.sync_copy(onesv, deg.at[dstv.at[j]], add=True)

        _ring_loop(gather_slice, srcv, (buf0, buf1, buf2),
                   (sem0, sem1, sem2), scat)

    for p in range(2):
        with_deg = p == 0
        pltpu.sync_copy(zer64, acc.at[stripe])
        if with_deg:
            pltpu.sync_copy(zer16, deg.at[stripe])

        @pl.when(c == 0)
        def _():
            pltpu.sync_copy(src4q.at[p, s], srcv)

        @pl.when(c == 1)
        def _():
            pltpu.sync_copy(src4q.at[2 + p, s], srcv)

        plsc.subcore_barrier()

        edge_loop(with_deg)

        plsc.subcore_barrier()

        @pl.when(c == 0)
        def _():
            pltpu.sync_copy(acc.at[stripe],
                            z.at[stripe, pl.ds(p * DSLAB, DSLAB)])

        @pl.when(c == 1)
        def _():
            pltpu.sync_copy(acc.at[stripe],
                            z.at[stripe, pl.ds(2 * DSLAB + p * DSLAB, DSLAB)])

        plsc.subcore_barrier()

    @pl.when(c == 0)
    def _():
        pltpu.sync_copy(deg.at[stripe], degbuf)

        def rbody(r, carry):
            d = degbuf[r, :]
            degbuf[r, :] = 1.0 / jnp.maximum(d, 1.0)
            return carry
        lax.fori_loop(0, ROWS_PER_SUB, rbody, 0)
        pltpu.sync_copy(degbuf, recip16.at[stripe])


_agg1 = pl.kernel(
    _agg1_body,
    out_type=(
        jax.ShapeDtypeStruct((N_PAD, DIN), jnp.float32),
        jax.ShapeDtypeStruct((N_PAD, 16), jnp.float32),
    ),
    mesh=_MESH,
    compiler_params=pltpu.CompilerParams(use_tc_tiling_on_sc=False),
    scratch_types=[
        pltpu.VMEM_SHARED((N_PAD, DSLAB), jnp.float32),
        pltpu.VMEM_SHARED((N_PAD, 16), jnp.float32),
        pltpu.VMEM((NCHUNK, CHUNK), jnp.int32),
        pltpu.VMEM((NCHUNK, CHUNK), jnp.int32),
        pltpu.VMEM((CHUNK, DSLAB), jnp.float32),
        pltpu.VMEM((CHUNK, DSLAB), jnp.float32),
        pltpu.VMEM((CHUNK, DSLAB), jnp.float32),
        pltpu.VMEM((CHUNK, 16), jnp.float32),
        pltpu.VMEM((ROWS_PER_SUB, 16), jnp.float32),
        pltpu.SemaphoreType.DMA,
        pltpu.SemaphoreType.DMA,
        pltpu.SemaphoreType.DMA,
    ],
)


def _agg2_body(y2, src2q, dst3, recip16, b2, zer32,
               out,
               acc, srcv, dstv, buf0, buf1, buf2, recipv, accbuf, b2v,
               sem0, sem1, sem2):
    c = lax.axis_index("c")
    s = lax.axis_index("s")
    row0 = s * ROWS_PER_SUB
    stripe = pl.ds(row0, ROWS_PER_SUB)

    pltpu.sync_copy(b2, b2v)
    pltpu.sync_copy(dst3.at[s], dstv)
    pltpu.sync_copy(zer32, acc.at[stripe])

    @pl.when(c == 0)
    def _():
        pltpu.sync_copy(src2q.at[0, s], srcv)

    @pl.when(c == 1)
    def _():
        pltpu.sync_copy(src2q.at[1, s], srcv)

    plsc.subcore_barrier()

    def edge_loop():
        def gather_slice(j):
            return y2.at[srcv.at[j]]

        def scat(j, buf):
            pltpu.sync_copy(buf, acc.at[dstv.at[j]], add=True)

        _ring_loop(gather_slice, srcv, (buf0, buf1, buf2),
                   (sem0, sem1, sem2), scat)

    edge_loop()

    plsc.subcore_barrier()

    pltpu.sync_copy(acc.at[stripe], accbuf)
    pltpu.sync_copy(recip16.at[stripe], recipv)
    b2a = b2v[pl.ds(c * DY, 16)]
    b2b = b2v[pl.ds(c * DY + 16, 16)]

    def rbody(r, carry):
        rv = recipv[r, :]
        accbuf[r, pl.ds(0, 16)] = accbuf[r, pl.ds(0, 16)] * rv + b2a
        accbuf[r, pl.ds(16, 16)] = accbuf[r, pl.ds(16, 16)] * rv + b2b
        return carry
    lax.fori_loop(0, ROWS_PER_SUB, rbody, 0)

    @pl.when(c == 0)
    def _():
        pltpu.sync_copy(accbuf, out.at[stripe, pl.ds(0, DY)])

    @pl.when(c == 1)
    def _():
        pltpu.sync_copy(accbuf, out.at[stripe, pl.ds(DY, DY)])


_agg2 = pl.kernel(
    _agg2_body,
    out_type=(
        jax.ShapeDtypeStruct((N_PAD, 2 * DY), jnp.float32),
    ),
    mesh=_MESH,
    compiler_params=pltpu.CompilerParams(use_tc_tiling_on_sc=False),
    scratch_types=[
        pltpu.VMEM_SHARED((N_PAD, DY), jnp.float32),
        pltpu.VMEM((NCHUNK, CHUNK), jnp.int32),
        pltpu.VMEM((NCHUNK, CHUNK), jnp.int32),
        pltpu.VMEM((CHUNK, DY), jnp.float32),
        pltpu.VMEM((CHUNK, DY), jnp.float32),
        pltpu.VMEM((CHUNK, DY), jnp.float32),
        pltpu.VMEM((ROWS_PER_SUB, 16), jnp.float32),
        pltpu.VMEM((ROWS_PER_SUB, DY), jnp.float32),
        pltpu.VMEM((2 * DY,), jnp.float32),
        pltpu.SemaphoreType.DMA,
        pltpu.SemaphoreType.DMA,
        pltpu.SemaphoreType.DMA,
    ],
)


def _mlp_body(z_ref, r16_ref, w1_ref, b1_ref, w2_ref, y_ref):
    z = z_ref[...]
    r = r16_ref[...][:, 0:1]
    h = jnp.dot(z * r, w1_ref[...], preferred_element_type=jnp.float32)
    h = jnp.maximum(h + b1_ref[...], 0.0)
    y_ref[...] = jnp.dot(h, w2_ref[...], preferred_element_type=jnp.float32)


def _mlp(z, recip16, W1, b1, W2):
    blk = 2048
    grid = (N_PAD // blk,)
    return pl.pallas_call(
        _mlp_body,
        grid=grid,
        in_specs=[
            pl.BlockSpec((blk, DIN), lambda i: (i, 0)),
            pl.BlockSpec((blk, 16), lambda i: (i, 0)),
            pl.BlockSpec((DIN, DIN), lambda i: (0, 0)),
            pl.BlockSpec((1, DIN), lambda i: (0, 0)),
            pl.BlockSpec((DIN, 2 * DY), lambda i: (0, 0)),
        ],
        out_specs=pl.BlockSpec((blk, 2 * DY), lambda i: (i, 0)),
        out_shape=jax.ShapeDtypeStruct((N_PAD, 2 * DY), jnp.float32),
    )(z, recip16, W1, b1, W2)


@jax.jit
def kernel(inputs, edge_index, W1, b1, W2, b2):
    src = edge_index[0].astype(jnp.int32)
    dst = edge_index[1].astype(jnp.int32)
    dst3 = dst.reshape(NSUB, NCHUNK, CHUNK)

    src4 = (src * 4).reshape(NSUB, NCHUNK, CHUNK)
    src4q = jnp.stack([src4, src4 + 1, src4 + 2, src4 + 3])
    src2 = (src * 2).reshape(NSUB, NCHUNK, CHUNK)
    src2q = jnp.stack([src2, src2 + 1])

    x4 = inputs.reshape(4 * N_NODES, DSLAB)

    zer64 = jnp.zeros((ROWS_PER_SUB, DSLAB), jnp.float32)
    zer16 = jnp.zeros((ROWS_PER_SUB, 16), jnp.float32)
    zer32 = jnp.zeros((ROWS_PER_SUB, DY), jnp.float32)
    ones16 = jnp.ones((CHUNK, 16), jnp.float32)

    z, recip16 = _agg1(x4, src4q, dst3, zer64, zer16, ones16)
    y = _mlp(z, recip16, W1, b1.reshape(1, -1), W2)
    y2 = y.reshape(2 * N_PAD, DY)
    out, = _agg2(y2, src2q, dst3, recip16, b2, zer32)
    return out[:N_NODES]


# speedup vs baseline: 1.0207x; 1.0207x over previous
import jax
import jax.numpy as jnp
from jax import lax
from jax.experimental import pallas as pl
from jax.experimental.pallas import tpu as pltpu
from jax.experimental.pallas import tpu_sc as plsc

N_NODES = 10000
N_PAD = 10240
N_EDGES = 160000
NCORE = 2
NSUB = 16
CHUNK = 200
EDGES_PER_SUB = N_EDGES // NSUB
NCHUNK = EDGES_PER_SUB // CHUNK
CHUNK2 = 400
NCHUNK2 = EDGES_PER_SUB // CHUNK2
NBUF = 3
ROWS_PER_SUB = N_PAD // NSUB
DIN = 256
DSLAB = 64
DY = 32

_MESH = plsc.VectorSubcoreMesh(core_axis_name="c", subcore_axis_name="s",
                               num_cores=NCORE, num_subcores=NSUB)


def _ring_loop(nchunk, gather_slice, srcv, bufs, sems, scat):
    pre = min(NBUF - 1, nchunk)
    main = ((nchunk - pre) // NBUF) * NBUF

    def fire(j, b):
        pltpu.async_copy(gather_slice(j), bufs[b], sems[b])

    def wait(j, b):
        pltpu.make_async_copy(gather_slice(j), bufs[b], sems[b]).wait()

    for j in range(pre):
        fire(j, j % NBUF)

    def body(i, carry):
        for b in range(NBUF):
            j = NBUF * i + b
            wait(j, b)
            fire(j + pre, (b + pre) % NBUF)
            scat(j, bufs[b])
        return carry
    lax.fori_loop(0, main // NBUF, body, 0)

    for j in range(main, nchunk):
        wait(j, j % NBUF)
        if j + pre < nchunk:
            fire(j + pre, (j + pre) % NBUF)
        scat(j, bufs[j % NBUF])


def _agg1_body(x4, src4q, dst3, zer64, zer16, ones16,
               z, recip16,
               acc, deg, srcv, dstv, buf0, buf1, buf2, onesv, degbuf,
               sem0, sem1, sem2):
    c = lax.axis_index("c")
    s = lax.axis_index("s")
    row0 = s * ROWS_PER_SUB
    stripe = pl.ds(row0, ROWS_PER_SUB)

    pltpu.sync_copy(ones16, onesv)
    pltpu.sync_copy(dst3.at[s], dstv)

    def edge_loop(with_deg):
        def gather_slice(j):
            return x4.at[srcv.at[j]]

        def scat(j, buf):
            pltpu.sync_copy(buf, acc.at[dstv.at[j]], add=True)
            if with_deg:
                pltpu.sync_copy(onesv, deg.at[dstv.at[j]], add=True)

        _ring_loop(NCHUNK, gather_slice, srcv, (buf0, buf1, buf2),
                   (sem0, sem1, sem2), scat)

    for p in range(2):
        with_deg = p == 0
        pltpu.sync_copy(zer64, acc.at[stripe])
        if with_deg:
            pltpu.sync_copy(zer16, deg.at[stripe])

        @pl.when(c == 0)
        def _():
            pltpu.sync_copy(src4q.at[p, s], srcv)

        @pl.when(c == 1)
        def _():
            pltpu.sync_copy(src4q.at[2 + p, s], srcv)

        plsc.subcore_barrier()

        edge_loop(with_deg)

        plsc.subcore_barrier()

        @pl.when(c == 0)
        def _():
            pltpu.sync_copy(acc.at[stripe],
                            z.at[stripe, pl.ds(p * DSLAB, DSLAB)])

        @pl.when(c == 1)
        def _():
            pltpu.sync_copy(acc.at[stripe],
                            z.at[stripe, pl.ds(2 * DSLAB + p * DSLAB, DSLAB)])

        plsc.subcore_barrier()

    @pl.when(c == 0)
    def _():
        pltpu.sync_copy(deg.at[stripe], degbuf)

        def rbody(r, carry):
            d = degbuf[r, :]
            degbuf[r, :] = 1.0 / jnp.maximum(d, 1.0)
            return carry
        lax.fori_loop(0, ROWS_PER_SUB, rbody, 0)
        pltpu.sync_copy(degbuf, recip16.at[stripe])


_agg1 = pl.kernel(
    _agg1_body,
    out_type=(
        jax.ShapeDtypeStruct((N_PAD, DIN), jnp.float32),
        jax.ShapeDtypeStruct((N_PAD, 16), jnp.float32),
    ),
    mesh=_MESH,
    compiler_params=pltpu.CompilerParams(use_tc_tiling_on_sc=False),
    scratch_types=[
        pltpu.VMEM_SHARED((N_PAD, DSLAB), jnp.float32),
        pltpu.VMEM_SHARED((N_PAD, 16), jnp.float32),
        pltpu.VMEM((NCHUNK, CHUNK), jnp.int32),
        pltpu.VMEM((NCHUNK, CHUNK), jnp.int32),
        pltpu.VMEM((CHUNK, DSLAB), jnp.float32),
        pltpu.VMEM((CHUNK, DSLAB), jnp.float32),
        pltpu.VMEM((CHUNK, DSLAB), jnp.float32),
        pltpu.VMEM((CHUNK, 16), jnp.float32),
        pltpu.VMEM((ROWS_PER_SUB, 16), jnp.float32),
        pltpu.SemaphoreType.DMA,
        pltpu.SemaphoreType.DMA,
        pltpu.SemaphoreType.DMA,
    ],
)


def _agg2_body(y2, src2q, dst3, recip16, b2, zer32,
               out,
               acc, srcv, dstv, buf0, buf1, buf2, recipv, accbuf, b2v,
               sem0, sem1, sem2):
    c = lax.axis_index("c")
    s = lax.axis_index("s")
    row0 = s * ROWS_PER_SUB
    stripe = pl.ds(row0, ROWS_PER_SUB)

    pltpu.sync_copy(b2, b2v)
    pltpu.sync_copy(dst3.at[s], dstv)
    pltpu.sync_copy(zer32, acc.at[stripe])

    @pl.when(c == 0)
    def _():
        pltpu.sync_copy(src2q.at[0, s], srcv)

    @pl.when(c == 1)
    def _():
        pltpu.sync_copy(src2q.at[1, s], srcv)

    plsc.subcore_barrier()

    def edge_loop():
        def gather_slice(j):
            return y2.at[srcv.at[j]]

        def scat(j, buf):
            pltpu.sync_copy(buf, acc.at[dstv.at[j]], add=True)

        _ring_loop(NCHUNK2, gather_slice, srcv, (buf0, buf1, buf2),
                   (sem0, sem1, sem2), scat)

    edge_loop()

    plsc.subcore_barrier()

    pltpu.sync_copy(acc.at[stripe], accbuf)
    pltpu.sync_copy(recip16.at[stripe], recipv)
    b2a = b2v[pl.ds(c * DY, 16)]
    b2b = b2v[pl.ds(c * DY + 16, 16)]

    def rbody(r, carry):
        rv = recipv[r, :]
        accbuf[r, pl.ds(0, 16)] = accbuf[r, pl.ds(0, 16)] * rv + b2a
        accbuf[r, pl.ds(16, 16)] = accbuf[r, pl.ds(16, 16)] * rv + b2b
        return carry
    lax.fori_loop(0, ROWS_PER_SUB, rbody, 0)

    @pl.when(c == 0)
    def _():
        pltpu.sync_copy(accbuf, out.at[stripe, pl.ds(0, DY)])

    @pl.when(c == 1)
    def _():
        pltpu.sync_copy(accbuf, out.at[stripe, pl.ds(DY, DY)])


_agg2 = pl.kernel(
    _agg2_body,
    out_type=(
        jax.ShapeDtypeStruct((N_PAD, 2 * DY), jnp.float32),
    ),
    mesh=_MESH,
    compiler_params=pltpu.CompilerParams(use_tc_tiling_on_sc=False),
    scratch_types=[
        pltpu.VMEM_SHARED((N_PAD, DY), jnp.float32),
        pltpu.VMEM((NCHUNK2, CHUNK2), jnp.int32),
        pltpu.VMEM((NCHUNK2, CHUNK2), jnp.int32),
        pltpu.VMEM((CHUNK2, DY), jnp.float32),
        pltpu.VMEM((CHUNK2, DY), jnp.float32),
        pltpu.VMEM((CHUNK2, DY), jnp.float32),
        pltpu.VMEM((ROWS_PER_SUB, 16), jnp.float32),
        pltpu.VMEM((ROWS_PER_SUB, DY), jnp.float32),
        pltpu.VMEM((2 * DY,), jnp.float32),
        pltpu.SemaphoreType.DMA,
        pltpu.SemaphoreType.DMA,
        pltpu.SemaphoreType.DMA,
    ],
)


def _mlp_body(z_ref, r16_ref, w1_ref, b1_ref, w2_ref, y_ref):
    z = z_ref[...]
    r = r16_ref[...][:, 0:1]
    h = jnp.dot(z * r, w1_ref[...], preferred_element_type=jnp.float32)
    h = jnp.maximum(h + b1_ref[...], 0.0)
    y_ref[...] = jnp.dot(h, w2_ref[...], preferred_element_type=jnp.float32)


def _mlp(z, recip16, W1, b1, W2):
    blk = 2048
    grid = (N_PAD // blk,)
    return pl.pallas_call(
        _mlp_body,
        grid=grid,
        in_specs=[
            pl.BlockSpec((blk, DIN), lambda i: (i, 0)),
            pl.BlockSpec((blk, 16), lambda i: (i, 0)),
            pl.BlockSpec((DIN, DIN), lambda i: (0, 0)),
            pl.BlockSpec((1, DIN), lambda i: (0, 0)),
            pl.BlockSpec((DIN, 2 * DY), lambda i: (0, 0)),
        ],
        out_specs=pl.BlockSpec((blk, 2 * DY), lambda i: (i, 0)),
        out_shape=jax.ShapeDtypeStruct((N_PAD, 2 * DY), jnp.float32),
    )(z, recip16, W1, b1, W2)


@jax.jit
def kernel(inputs, edge_index, W1, b1, W2, b2):
    src = edge_index[0].astype(jnp.int32)
    dst = edge_index[1].astype(jnp.int32)
    dst3 = dst.reshape(NSUB, NCHUNK, CHUNK)

    src4 = (src * 4).reshape(NSUB, NCHUNK, CHUNK)
    src4q = jnp.stack([src4, src4 + 1, src4 + 2, src4 + 3])
    src2 = (src * 2).reshape(NSUB, NCHUNK2, CHUNK2)
    src2q = jnp.stack([src2, src2 + 1])
    dst3b = dst.reshape(NSUB, NCHUNK2, CHUNK2)

    x4 = inputs.reshape(4 * N_NODES, DSLAB)

    zer64 = jnp.zeros((ROWS_PER_SUB, DSLAB), jnp.float32)
    zer16 = jnp.zeros((ROWS_PER_SUB, 16), jnp.float32)
    zer32 = jnp.zeros((ROWS_PER_SUB, DY), jnp.float32)
    ones16 = jnp.ones((CHUNK, 16), jnp.float32)

    z, recip16 = _agg1(x4, src4q, dst3, zer64, zer16, ones16)
    y = _mlp(z, recip16, W1, b1.reshape(1, -1), W2)
    y2 = y.reshape(2 * N_PAD, DY)
    out, = _agg2(y2, src2q, dst3b, recip16, b2, zer32)
    return out[:N_NODES]


# speedup vs baseline: 1.0439x; 1.0228x over previous
import jax
import jax.numpy as jnp
from jax import lax
from jax.experimental import pallas as pl
from jax.experimental.pallas import tpu as pltpu
from jax.experimental.pallas import tpu_sc as plsc

N_NODES = 10000
N_PAD = 10240
N_EDGES = 160000
NCORE = 2
NSUB = 16
CHUNK = 200
EDGES_PER_SUB = N_EDGES // NSUB
NCHUNK = EDGES_PER_SUB // CHUNK
CHUNK2 = 400
NCHUNK2 = EDGES_PER_SUB // CHUNK2
NBUF = 3
ROWS_PER_SUB = N_PAD // NSUB
DIN = 256
DSLAB = 64
DY = 32

_MESH = plsc.VectorSubcoreMesh(core_axis_name="c", subcore_axis_name="s",
                               num_cores=NCORE, num_subcores=NSUB)


def _ring_loop(nchunk, gather_slice, srcv, bufs, sems, scat):
    pre = min(NBUF - 1, nchunk)
    main = ((nchunk - pre) // NBUF) * NBUF

    def fire(j, b):
        pltpu.async_copy(gather_slice(j), bufs[b], sems[b])

    def wait(j, b):
        pltpu.make_async_copy(gather_slice(j), bufs[b], sems[b]).wait()

    for j in range(pre):
        fire(j, j % NBUF)

    def body(i, carry):
        for b in range(NBUF):
            j = NBUF * i + b
            wait(j, b)
            fire(j + pre, (b + pre) % NBUF)
            scat(j, bufs[b])
        return carry
    lax.fori_loop(0, main // NBUF, body, 0)

    for j in range(main, nchunk):
        wait(j, j % NBUF)
        if j + pre < nchunk:
            fire(j + pre, (j + pre) % NBUF)
        scat(j, bufs[j % NBUF])


def _agg1_body(x4, src4q, dst3, zer64, zer16, ones16,
               z, recip16,
               acc, deg, srcv, dstv, buf0, buf1, buf2, onesv, degbuf,
               sem0, sem1, sem2):
    c = lax.axis_index("c")
    s = lax.axis_index("s")
    row0 = s * ROWS_PER_SUB
    stripe = pl.ds(row0, ROWS_PER_SUB)

    pltpu.sync_copy(ones16, onesv)
    pltpu.sync_copy(dst3.at[s], dstv)

    def edge_loop(with_deg):
        def gather_slice(j):
            return x4.at[srcv.at[j]]

        def scat(j, buf):
            pltpu.sync_copy(buf, acc.at[dstv.at[j]], add=True)
            if with_deg:
                pltpu.sync_copy(onesv, deg.at[dstv.at[j]], add=True)

        _ring_loop(NCHUNK, gather_slice, srcv, (buf0, buf1, buf2),
                   (sem0, sem1, sem2), scat)

    for p in range(2):
        with_deg = p == 0
        pltpu.sync_copy(zer64, acc.at[stripe])
        if with_deg:
            pltpu.sync_copy(zer16, deg.at[stripe])

        @pl.when(c == 0)
        def _():
            pltpu.sync_copy(src4q.at[p, s], srcv)

        @pl.when(c == 1)
        def _():
            pltpu.sync_copy(src4q.at[2 + p, s], srcv)

        plsc.subcore_barrier()

        edge_loop(with_deg)

        plsc.subcore_barrier()

        @pl.when(c == 0)
        def _():
            pltpu.sync_copy(acc.at[stripe],
                            z.at[stripe, pl.ds(p * DSLAB, DSLAB)])

        @pl.when(c == 1)
        def _():
            pltpu.sync_copy(acc.at[stripe],
                            z.at[stripe, pl.ds(2 * DSLAB + p * DSLAB, DSLAB)])

        plsc.subcore_barrier()

    @pl.when(c == 0)
    def _():
        pltpu.sync_copy(deg.at[stripe], degbuf)

        def rbody(r, carry):
            d = degbuf[r, :]
            degbuf[r, :] = 1.0 / jnp.maximum(d, 1.0)
            return carry
        lax.fori_loop(0, ROWS_PER_SUB, rbody, 0)
        pltpu.sync_copy(degbuf, recip16.at[stripe])


_agg1 = pl.kernel(
    _agg1_body,
    out_type=(
        jax.ShapeDtypeStruct((N_PAD, DIN), jnp.float32),
        jax.ShapeDtypeStruct((N_PAD, 16), jnp.float32),
    ),
    mesh=_MESH,
    compiler_params=pltpu.CompilerParams(use_tc_tiling_on_sc=False),
    scratch_types=[
        pltpu.VMEM_SHARED((N_PAD, DSLAB), jnp.float32),
        pltpu.VMEM_SHARED((N_PAD, 16), jnp.float32),
        pltpu.VMEM((NCHUNK, CHUNK), jnp.int32),
        pltpu.VMEM((NCHUNK, CHUNK), jnp.int32),
        pltpu.VMEM((CHUNK, DSLAB), jnp.float32),
        pltpu.VMEM((CHUNK, DSLAB), jnp.float32),
        pltpu.VMEM((CHUNK, DSLAB), jnp.float32),
        pltpu.VMEM((CHUNK, 16), jnp.float32),
        pltpu.VMEM((ROWS_PER_SUB, 16), jnp.float32),
        pltpu.SemaphoreType.DMA,
        pltpu.SemaphoreType.DMA,
        pltpu.SemaphoreType.DMA,
    ],
)


def _agg2_body(y2, src2q, dst3, recip16, b2, zer32,
               out,
               acc, srcv, dstv, buf0, buf1, buf2, recipv, accbuf, b2v,
               sem0, sem1, sem2):
    c = lax.axis_index("c")
    s = lax.axis_index("s")
    row0 = s * ROWS_PER_SUB
    stripe = pl.ds(row0, ROWS_PER_SUB)

    pltpu.sync_copy(b2, b2v)
    pltpu.sync_copy(dst3.at[s], dstv)
    pltpu.sync_copy(zer32, acc.at[stripe])

    @pl.when(c == 0)
    def _():
        pltpu.sync_copy(src2q.at[0, s], srcv)

    @pl.when(c == 1)
    def _():
        pltpu.sync_copy(src2q.at[1, s], srcv)

    plsc.subcore_barrier()

    def edge_loop():
        def gather_slice(j):
            return y2.at[srcv.at[j]]

        def scat(j, buf):
            pltpu.sync_copy(buf, acc.at[dstv.at[j]], add=True)

        _ring_loop(NCHUNK2, gather_slice, srcv, (buf0, buf1, buf2),
                   (sem0, sem1, sem2), scat)

    edge_loop()

    plsc.subcore_barrier()

    pltpu.sync_copy(acc.at[stripe], accbuf)
    pltpu.sync_copy(recip16.at[stripe], recipv)
    b2a = b2v[pl.ds(c * DY, 16)]
    b2b = b2v[pl.ds(c * DY + 16, 16)]

    def rbody(r, carry):
        rv = recipv[r, :]
        accbuf[r, pl.ds(0, 16)] = accbuf[r, pl.ds(0, 16)] * rv + b2a
        accbuf[r, pl.ds(16, 16)] = accbuf[r, pl.ds(16, 16)] * rv + b2b
        return carry
    lax.fori_loop(0, ROWS_PER_SUB, rbody, 0)

    @pl.when(c == 0)
    def _():
        @pl.when(s < NSUB - 1)
        def _():
            pltpu.sync_copy(accbuf, out.at[stripe, pl.ds(0, DY)])

        @pl.when(s == NSUB - 1)
        def _():
            pltpu.sync_copy(
                accbuf.at[pl.ds(0, N_NODES - (NSUB - 1) * ROWS_PER_SUB)],
                out.at[pl.ds((NSUB - 1) * ROWS_PER_SUB,
                             N_NODES - (NSUB - 1) * ROWS_PER_SUB),
                       pl.ds(0, DY)])

    @pl.when(c == 1)
    def _():
        @pl.when(s < NSUB - 1)
        def _():
            pltpu.sync_copy(accbuf, out.at[stripe, pl.ds(DY, DY)])

        @pl.when(s == NSUB - 1)
        def _():
            pltpu.sync_copy(
                accbuf.at[pl.ds(0, N_NODES - (NSUB - 1) * ROWS_PER_SUB)],
                out.at[pl.ds((NSUB - 1) * ROWS_PER_SUB,
                             N_NODES - (NSUB - 1) * ROWS_PER_SUB),
                       pl.ds(DY, DY)])


_agg2 = pl.kernel(
    _agg2_body,
    out_type=(
        jax.ShapeDtypeStruct((N_NODES, 2 * DY), jnp.float32),
    ),
    mesh=_MESH,
    compiler_params=pltpu.CompilerParams(use_tc_tiling_on_sc=False),
    scratch_types=[
        pltpu.VMEM_SHARED((N_PAD, DY), jnp.float32),
        pltpu.VMEM((NCHUNK2, CHUNK2), jnp.int32),
        pltpu.VMEM((NCHUNK2, CHUNK2), jnp.int32),
        pltpu.VMEM((CHUNK2, DY), jnp.float32),
        pltpu.VMEM((CHUNK2, DY), jnp.float32),
        pltpu.VMEM((CHUNK2, DY), jnp.float32),
        pltpu.VMEM((ROWS_PER_SUB, 16), jnp.float32),
        pltpu.VMEM((ROWS_PER_SUB, DY), jnp.float32),
        pltpu.VMEM((2 * DY,), jnp.float32),
        pltpu.SemaphoreType.DMA,
        pltpu.SemaphoreType.DMA,
        pltpu.SemaphoreType.DMA,
    ],
)


def _mlp_body(z_ref, r16_ref, w1_ref, b1_ref, w2_ref, y_ref):
    z = z_ref[...]
    r = r16_ref[...][:, 0:1]
    h = jnp.dot(z * r, w1_ref[...], preferred_element_type=jnp.float32)
    h = jnp.maximum(h + b1_ref[...], 0.0)
    y_ref[...] = jnp.dot(h, w2_ref[...], preferred_element_type=jnp.float32)


def _mlp(z, recip16, W1, b1, W2):
    blk = 2048
    grid = (N_PAD // blk,)
    return pl.pallas_call(
        _mlp_body,
        grid=grid,
        in_specs=[
            pl.BlockSpec((blk, DIN), lambda i: (i, 0)),
            pl.BlockSpec((blk, 16), lambda i: (i, 0)),
            pl.BlockSpec((DIN, DIN), lambda i: (0, 0)),
            pl.BlockSpec((1, DIN), lambda i: (0, 0)),
            pl.BlockSpec((DIN, 2 * DY), lambda i: (0, 0)),
        ],
        out_specs=pl.BlockSpec((blk, 2 * DY), lambda i: (i, 0)),
        out_shape=jax.ShapeDtypeStruct((N_PAD, 2 * DY), jnp.float32),
    )(z, recip16, W1, b1, W2)


@jax.jit
def kernel(inputs, edge_index, W1, b1, W2, b2):
    src = edge_index[0].astype(jnp.int32)
    dst = edge_index[1].astype(jnp.int32)
    dst3 = dst.reshape(NSUB, NCHUNK, CHUNK)

    src4 = (src * 4).reshape(NSUB, NCHUNK, CHUNK)
    src4q = jnp.stack([src4, src4 + 1, src4 + 2, src4 + 3])
    src2 = (src * 2).reshape(NSUB, NCHUNK2, CHUNK2)
    src2q = jnp.stack([src2, src2 + 1])
    dst3b = dst.reshape(NSUB, NCHUNK2, CHUNK2)

    x4 = inputs.reshape(4 * N_NODES, DSLAB)

    zer64 = jnp.zeros((ROWS_PER_SUB, DSLAB), jnp.float32)
    zer16 = jnp.zeros((ROWS_PER_SUB, 16), jnp.float32)
    zer32 = jnp.zeros((ROWS_PER_SUB, DY), jnp.float32)
    ones16 = jnp.ones((CHUNK, 16), jnp.float32)

    z, recip16 = _agg1(x4, src4q, dst3, zer64, zer16, ones16)
    y = _mlp(z, recip16, W1, b1.reshape(1, -1), W2)
    y2 = y.reshape(2 * N_PAD, DY)
    out, = _agg2(y2, src2q, dst3b, recip16, b2, zer32)
    return out
